# Initial kernel scaffold; baseline (speedup 1.0000x reference)
import jax
import jax.numpy as jnp
from jax.experimental import pallas as pl


def kernel(neiberm, neibern, u, v, u_weight, v_weight, att, M, N, duser, ditemid):
    raise NotImplementedError("write your pallas kernel here")


# speedup vs baseline: 5.8264x; 5.8264x over previous
import functools
import numpy as np
import jax
import jax.numpy as jnp
from jax import lax
from jax.experimental import pallas as pl
from jax.experimental.pallas import tpu as pltpu
from jax.experimental.pallas import tpu_sc as plsc

_K = 4
_A_COEF = 1.0
_B_COEF = 0.01
_DIM = 128
_MNBR = 16
_DD = _DIM // _K
_ZW = _MNBR * _DIM

_F32 = jnp.float32

_A2_NP = np.zeros((_ZW, _MNBR * _K), np.float32)
for _m in range(_MNBR):
    for _kk in range(_K):
        _A2_NP[_m * _DIM + _kk * _DD:_m * _DIM + (_kk + 1) * _DD, _m * _K + _kk] = 1.0
_A2T_NP = np.ascontiguousarray(_A2_NP.T)
_REP_NP = np.zeros((_MNBR, _MNBR * _K), np.float32)
for _m in range(_MNBR):
    _REP_NP[_m, _m * _K:(_m + 1) * _K] = 1.0
_F128_NP = np.tile(np.eye(_DIM, dtype=np.float32), (_MNBR, 1))


def _sc_gather(table, idx, nrows, chunk, nfire):
    nworkers = 32
    rw = nrows // nworkers
    nch = rw // chunk
    ngrp = nch // nfire
    assert rw * nworkers == nrows and chunk * nch == rw and nfire * ngrp == nch
    mesh = plsc.VectorSubcoreMesh(core_axis_name="c", subcore_axis_name="s")

    @functools.partial(
        pl.kernel,
        mesh=mesh,
        out_type=jax.ShapeDtypeStruct((nrows, _DIM), _F32),
        scratch_types=[
            pltpu.VMEM((rw,), jnp.int32),
            pltpu.VMEM((nfire, chunk, _DIM), _F32),
            pltpu.SemaphoreType.DMA,
        ],
    )
    def gk(table_hbm, idx_hbm, out_hbm, idx_v, rows_v, sem):
        wid = lax.axis_index("s") * 2 + lax.axis_index("c")
        base = wid * rw
        pltpu.sync_copy(idx_hbm.at[pl.ds(base, rw)], idx_v)

        def grp(g, carry):
            handles = []
            for b in range(nfire):
                off = (g * nfire + b) * chunk
                handles.append(
                    pltpu.async_copy(
                        table_hbm.at[idx_v.at[pl.ds(off, chunk)]],
                        rows_v.at[b],
                        sem,
                    )
                )
            for b in range(nfire):
                handles[b].wait()
                off = (g * nfire + b) * chunk
                pltpu.sync_copy(rows_v.at[b], out_hbm.at[pl.ds(base + off, chunk)])
            return carry

        lax.fori_loop(0, ngrp, grp, 0)

    return gk(table, idx)


def _routing_body(z_ref, x_ref, a1_ref, a2_ref, a2t_ref, rep_ref, f128_ref, o_ref):
    z = z_ref[...]
    x = x_ref[...]
    dn = (((1,), (0,)), ((), ()))
    ez = lax.dot_general(z, a1_ref[...], dn, preferred_element_type=_F32)
    emax = jnp.max(ez, axis=1, keepdims=True)
    ee = jnp.exp(ez - emax)
    w = ee / jnp.sum(ee, axis=1, keepdims=True)
    s64 = lax.dot_general(z, a2_ref[...], dn, preferred_element_type=_F32)
    w64 = lax.dot_general(w, rep_ref[...], dn, preferred_element_type=_F32)
    coef = w64 * s64
    cexp = lax.dot_general(coef, a2t_ref[...], dn, preferred_element_type=_F32)
    fold = lax.dot_general(cexp * z, f128_ref[...], dn, preferred_element_type=_F32)
    o_ref[...] = x + _A_COEF * fold


def _routing_call(z2, x, a1):
    n = x.shape[0]
    br = 200
    grid = (n // br,)
    return pl.pallas_call(
        _routing_body,
        grid=grid,
        in_specs=[
            pl.BlockSpec((br, _ZW), lambda i: (i, 0)),
            pl.BlockSpec((br, _DIM), lambda i: (i, 0)),
            pl.BlockSpec((_ZW, _MNBR), lambda i: (0, 0)),
            pl.BlockSpec((_ZW, _MNBR * _K), lambda i: (0, 0)),
            pl.BlockSpec((_MNBR * _K, _ZW), lambda i: (0, 0)),
            pl.BlockSpec((_MNBR, _MNBR * _K), lambda i: (0, 0)),
            pl.BlockSpec((_ZW, _DIM), lambda i: (0, 0)),
        ],
        out_specs=pl.BlockSpec((br, _DIM), lambda i: (i, 0)),
        out_shape=jax.ShapeDtypeStruct((n, _DIM), _F32),
    )(z2, x, a1, jnp.asarray(_A2_NP), jnp.asarray(_A2T_NP), jnp.asarray(_REP_NP),
      jnp.asarray(_F128_NP))


def _pr_body(a_ref, b_ref, pr_ref, pbm_ref):
    prv = jnp.sum(a_ref[...] * b_ref[...], axis=1, keepdims=True)
    pr_ref[...] = prv
    ls = jnp.minimum(prv, 0.0) - jnp.log(1.0 + jnp.exp(-jnp.abs(prv)))
    pbm_ref[...] = jnp.sum(ls, axis=0, keepdims=True) / prv.shape[0]


def _pr_call(a, b):
    nb = a.shape[0]
    return pl.pallas_call(
        _pr_body,
        in_specs=[
            pl.BlockSpec((nb, _DIM), lambda: (0, 0)),
            pl.BlockSpec((nb, _DIM), lambda: (0, 0)),
        ],
        out_specs=[
            pl.BlockSpec((nb, 1), lambda: (0, 0)),
            pl.BlockSpec((1, 1), lambda: (0, 0)),
        ],
        out_shape=[
            jax.ShapeDtypeStruct((nb, 1), _F32),
            jax.ShapeDtypeStruct((1, 1), _F32),
        ],
    )(a, b)


def _kl_body(eb_ref, et_ref, m_ref, o_ref):
    i = pl.program_id(0)
    um = lax.dot_general(eb_ref[...], et_ref[...], (((1,), (0,)), ((), ())),
                         preferred_element_type=_F32)
    tu = 1.0 / (1.0 + jnp.exp(-um))
    lse = jnp.log(jnp.sum(jnp.exp(tu), axis=1, keepdims=True))
    mb = m_ref[...]
    r = jnp.sum(mb, axis=1, keepdims=True)
    a = mb / r
    amax = jnp.max(a, axis=1, keepdims=True)
    pe = jnp.exp(a - amax)
    s = jnp.sum(pe, axis=1, keepdims=True)
    t1 = jnp.sum(pe * a, axis=1, keepdims=True)
    t2 = jnp.sum(pe * tu, axis=1, keepdims=True)
    kl_row = (t1 - t2) / s - amax - jnp.log(s) + lse
    bs = jnp.sum(kl_row, axis=0, keepdims=True)

    @pl.when(i == 0)
    def _():
        o_ref[...] = jnp.zeros((1, 1), _F32)

    o_ref[...] += bs


def _kl_call(mat, emb, embt):
    n = emb.shape[0]
    br = 200
    return pl.pallas_call(
        _kl_body,
        grid=(n // br,),
        in_specs=[
            pl.BlockSpec((br, _DIM), lambda i: (i, 0)),
            pl.BlockSpec((_DIM, n), lambda i: (0, 0)),
            pl.BlockSpec((br, n), lambda i: (i, 0)),
        ],
        out_specs=pl.BlockSpec((1, 1), lambda i: (0, 0)),
        out_shape=jax.ShapeDtypeStruct((1, 1), _F32),
    )(emb, embt, mat)


def kernel(neiberm, neibern, u, v, u_weight, v_weight, att, M, N, duser, ditemid):
    mlen = u_weight.shape[0]
    ui_emb = jnp.concatenate([u_weight, v_weight], axis=0)
    nbs = jnp.concatenate(
        [neiberm.reshape(-1), neibern.reshape(-1) + mlen]).astype(jnp.int32)

    z_flat = _sc_gather(ui_emb, nbs, nbs.shape[0], 40, 5)
    z2 = z_flat.reshape(ui_emb.shape[0], _ZW)

    a1 = jnp.kron(jnp.eye(_MNBR, dtype=_F32), att[_DIM:].astype(_F32))
    out_emb = _routing_call(z2, ui_emb, a1)
    user_emb, item_emb = out_emb[:mlen], out_emb[mlen:]

    pidx = jnp.concatenate(
        [duser.astype(jnp.int32), ditemid.astype(jnp.int32) + mlen])
    prrows = _sc_gather(out_emb, pidx, pidx.shape[0], 32, 8)
    nb = duser.shape[0]
    pr2, pbm = _pr_call(prrows[:nb], prrows[nb:])
    pr = pr2[:, 0]

    kl_u = _kl_call(M, user_emb, user_emb.T)
    kl_v = _kl_call(N, item_emb, item_emb.T)
    loss = -pbm[0, 0] + _B_COEF * kl_u[0, 0] + _B_COEF * kl_v[0, 0]
    return (user_emb, item_emb, loss, pr)
